# baseline (speedup 1.0000x reference)
import jax
import jax.numpy as jnp
from jax.experimental import pallas as pl
from jax.experimental.pallas import tpu as pltpu

N = 2048
D = 128
H = 8
DH = D // H
L = 2
FF = 256
NRBF = 8
NAA = 21
BLK = 256
IB = N // BLK
JB = N // BLK


def _ln(h):
    mu = jnp.mean(h, axis=-1, keepdims=True)
    hc = h - mu
    var = jnp.mean(hc * hc, axis=-1, keepdims=True)
    return hc / jnp.sqrt(var + 1e-5)


def _prep_body(apx_ref, apy_ref, apz_ref, mask_ref, aa_ref, wx_ref, wy_ref,
               wz_ref, emb_ref, bpos_ref, h0_ref, a_ref, b_ref):
    m = (mask_ref[...] > 0.5).astype(jnp.float32)
    apx = apx_ref[...]
    apy = apy_ref[...]
    apz = apz_ref[...]
    cax = apx[:, 1:2]
    cay = apy[:, 1:2]
    caz = apz[:, 1:2]
    h = jnp.dot((apx - cax) * m, wx_ref[...], preferred_element_type=jnp.float32)
    h += jnp.dot((apy - cay) * m, wy_ref[...], preferred_element_type=jnp.float32)
    h += jnp.dot((apz - caz) * m, wz_ref[...], preferred_element_type=jnp.float32)
    ai = jnp.clip(aa_ref[...].astype(jnp.int32), 0, NAA - 1)
    iota = jax.lax.broadcasted_iota(jnp.int32, (N, NAA), 1)
    onehot = (ai == iota).astype(jnp.float32)
    h += jnp.dot(onehot, emb_ref[...], preferred_element_type=jnp.float32)
    h0_ref[...] = h + bpos_ref[...]
    n2 = cax * cax + cay * cay + caz * caz
    ones = jnp.ones_like(n2)
    zeros = jnp.zeros_like(n2)
    a_ref[...] = jnp.concatenate(
        [-2.0 * cax, -2.0 * cay, -2.0 * caz, n2, ones, zeros, zeros, zeros], axis=1)
    b_ref[...] = jnp.concatenate(
        [cax, cay, caz, ones, n2, zeros, zeros, zeros], axis=1)


def _main_body(h0_ref, a_ref, b_ref, ci_ref, cj_ref, wq_ref, wk_ref, wv_ref,
               wo_ref, wpair_ref, w1_ref, w2_ref, out_ref,
               h_scr, q_scr, k_scr, v_scr, m_scr, s_scr, acc_scr):
    l = pl.program_id(0)
    ib = pl.program_id(1)
    jb = pl.program_id(2)

    @pl.when((l == 0) & (ib == 0) & (jb == 0))
    def _init():
        h_scr[...] = h0_ref[...]

    @pl.when((ib == 0) & (jb == 0))
    def _qkv():
        hn = _ln(h_scr[...])
        q_scr[...] = jnp.dot(hn, wq_ref[0], preferred_element_type=jnp.float32)
        k_scr[...] = jnp.dot(hn, wk_ref[0], preferred_element_type=jnp.float32)
        v_scr[...] = jnp.dot(hn, wv_ref[0], preferred_element_type=jnp.float32)

    @pl.when(jb == 0)
    def _reset():
        m_scr[...] = jnp.full((BLK, H), -1e30, dtype=jnp.float32)
        s_scr[...] = jnp.zeros((BLK, H), dtype=jnp.float32)
        acc_scr[...] = jnp.zeros((BLK, D), dtype=jnp.float32)

    d2 = jax.lax.dot_general(a_ref[...], b_ref[...], (((1,), (1,)), ((), ())),
                             preferred_element_type=jnp.float32)
    d = jnp.sqrt(jnp.maximum(d2, 0.0) + 1e-8)
    same = (ci_ref[...] == cj_ref[0:1, :]).astype(jnp.float32)
    biases = [same * wpair_ref[l, NRBF, h] for h in range(H)]
    for c in range(NRBF):
        cc = 2.0 * c / (NRBF - 1)
        r = jnp.exp((d - cc) * (d - cc) * -8.0)
        for h in range(H):
            biases[h] += r * wpair_ref[l, c, h]

    q = q_scr[pl.ds(ib * BLK, BLK), :]
    k = k_scr[pl.ds(jb * BLK, BLK), :]
    v = v_scr[pl.ds(jb * BLK, BLK), :]
    scale = 1.0 / (DH ** 0.5)
    for h in range(H):
        sl = slice(h * DH, (h + 1) * DH)
        logits = jax.lax.dot_general(q[:, sl] * scale, k[:, sl],
                                     (((1,), (1,)), ((), ())),
                                     preferred_element_type=jnp.float32)
        logits += biases[h]
        m_old = m_scr[:, h:h + 1]
        m_new = jnp.maximum(m_old, jnp.max(logits, axis=1, keepdims=True))
        p = jnp.exp(logits - m_new)
        corr = jnp.exp(m_old - m_new)
        s_scr[:, h:h + 1] = s_scr[:, h:h + 1] * corr + jnp.sum(p, axis=1, keepdims=True)
        acc_scr[:, sl] = acc_scr[:, sl] * corr + jnp.dot(
            p, v[:, sl], preferred_element_type=jnp.float32)
        m_scr[:, h:h + 1] = m_new

    @pl.when(jb == JB - 1)
    def _final():
        o = jnp.concatenate(
            [acc_scr[:, h * DH:(h + 1) * DH] / s_scr[:, h:h + 1] for h in range(H)],
            axis=1)
        rows = pl.ds(ib * BLK, BLK)
        h1 = h_scr[rows, :] + jnp.dot(o, wo_ref[0], preferred_element_type=jnp.float32)
        ffin = jax.nn.relu(jnp.dot(_ln(h1), w1_ref[0], preferred_element_type=jnp.float32))
        h2 = h1 + jnp.dot(ffin, w2_ref[0], preferred_element_type=jnp.float32)
        h_scr[rows, :] = h2

    @pl.when((jb == JB - 1) & (l == L - 1))
    def _out():
        out_ref[...] = h_scr[pl.ds(ib * BLK, BLK), :]


def kernel(x, edge_index, edge_attr, aa_embed, w_pos, b_pos, wq, wk, wv, wo,
           w_pair, w1, w2):
    apx = x[:, 0:42:3]
    apy = x[:, 1:42:3]
    apz = x[:, 2:42:3]
    mask = x[:, 45:59]
    aa_col = x[:, 42:43]
    chain = x[:, 44].astype(jnp.int32)
    ci = chain.reshape(N, 1)
    cj8 = jnp.broadcast_to(chain.reshape(1, N), (8, N))
    wx = w_pos[0::3]
    wy = w_pos[1::3]
    wz = w_pos[2::3]
    bpos = b_pos.reshape(1, D)

    h0, a_mat, b_mat = pl.pallas_call(
        _prep_body,
        out_shape=[
            jax.ShapeDtypeStruct((N, D), jnp.float32),
            jax.ShapeDtypeStruct((N, 8), jnp.float32),
            jax.ShapeDtypeStruct((N, 8), jnp.float32),
        ],
    )(apx, apy, apz, mask, aa_col, wx, wy, wz, aa_embed, bpos)

    grid = (L, IB, JB)
    out = pl.pallas_call(
        _main_body,
        grid=grid,
        in_specs=[
            pl.BlockSpec((N, D), lambda l, i, j: (0, 0)),
            pl.BlockSpec((BLK, 8), lambda l, i, j: (i, 0)),
            pl.BlockSpec((BLK, 8), lambda l, i, j: (j, 0)),
            pl.BlockSpec((BLK, 1), lambda l, i, j: (i, 0)),
            pl.BlockSpec((8, BLK), lambda l, i, j: (0, j)),
            pl.BlockSpec((1, D, D), lambda l, i, j: (l, 0, 0)),
            pl.BlockSpec((1, D, D), lambda l, i, j: (l, 0, 0)),
            pl.BlockSpec((1, D, D), lambda l, i, j: (l, 0, 0)),
            pl.BlockSpec((1, D, D), lambda l, i, j: (l, 0, 0)),
            pl.BlockSpec(memory_space=pltpu.SMEM),
            pl.BlockSpec((1, D, FF), lambda l, i, j: (l, 0, 0)),
            pl.BlockSpec((1, FF, D), lambda l, i, j: (l, 0, 0)),
        ],
        out_specs=pl.BlockSpec((BLK, D), lambda l, i, j: (i, 0)),
        out_shape=jax.ShapeDtypeStruct((N, D), jnp.float32),
        scratch_shapes=[
            pltpu.VMEM((N, D), jnp.float32),
            pltpu.VMEM((N, D), jnp.float32),
            pltpu.VMEM((N, D), jnp.float32),
            pltpu.VMEM((N, D), jnp.float32),
            pltpu.VMEM((BLK, H), jnp.float32),
            pltpu.VMEM((BLK, H), jnp.float32),
            pltpu.VMEM((BLK, D), jnp.float32),
        ],
        compiler_params=pltpu.CompilerParams(
            dimension_semantics=("arbitrary", "arbitrary", "arbitrary")),
    )(h0, a_mat, b_mat, ci, cj8, wq, wk, wv, wo, w_pair, w1, w2)
    return out.reshape(1, N, D)


# speedup vs baseline: 2.9704x; 2.9704x over previous
import jax
import jax.numpy as jnp
from jax.experimental import pallas as pl
from jax.experimental.pallas import tpu as pltpu

N = 2048
D = 128
H = 8
DH = D // H
L = 2
FF = 256
NRBF = 8
NAA = 21
BLK = 256
IB = N // BLK
JB = N // BLK


def _ln(h):
    mu = jnp.mean(h, axis=-1, keepdims=True)
    hc = h - mu
    var = jnp.mean(hc * hc, axis=-1, keepdims=True)
    return hc / jnp.sqrt(var + 1e-5)


def _prep_body(apx_ref, apy_ref, apz_ref, mask_ref, aa_ref, wx_ref, wy_ref,
               wz_ref, emb_ref, bpos_ref, h0_ref, a_ref, b_ref):
    m = (mask_ref[...] > 0.5).astype(jnp.float32)
    apx = apx_ref[...]
    apy = apy_ref[...]
    apz = apz_ref[...]
    cax = apx[:, 1:2]
    cay = apy[:, 1:2]
    caz = apz[:, 1:2]
    h = jnp.dot((apx - cax) * m, wx_ref[...], preferred_element_type=jnp.float32)
    h += jnp.dot((apy - cay) * m, wy_ref[...], preferred_element_type=jnp.float32)
    h += jnp.dot((apz - caz) * m, wz_ref[...], preferred_element_type=jnp.float32)
    ai = jnp.clip(aa_ref[...].astype(jnp.int32), 0, NAA - 1)
    iota = jax.lax.broadcasted_iota(jnp.int32, (N, NAA), 1)
    onehot = (ai == iota).astype(jnp.float32)
    h += jnp.dot(onehot, emb_ref[...], preferred_element_type=jnp.float32)
    h0_ref[...] = h + bpos_ref[...]
    n2 = cax * cax + cay * cay + caz * caz
    ones = jnp.ones_like(n2)
    zeros = jnp.zeros_like(n2)
    a_ref[...] = jnp.concatenate(
        [-2.0 * cax, -2.0 * cay, -2.0 * caz, n2, ones, zeros, zeros, zeros], axis=1)
    b_ref[...] = jnp.concatenate(
        [cax, cay, caz, ones, n2, zeros, zeros, zeros], axis=1)


def _pre_body(h_ref, wq_ref, wk_ref, wv_ref, wpair_ref,
              q_ref, k_ref, vext_ref, mb_ref):
    hn = _ln(h_ref[...])
    q = jnp.dot(hn, wq_ref[...], preferred_element_type=jnp.float32)
    k = jnp.dot(hn, wk_ref[...], preferred_element_type=jnp.float32)
    v = jnp.dot(hn, wv_ref[...], preferred_element_type=jnp.float32)
    q_ref[...] = q
    k_ref[...] = k
    ones = jnp.ones((N, 1), dtype=jnp.float32)
    zer = jnp.zeros((N, DH - 1), dtype=jnp.float32)
    pieces = []
    for h in range(H):
        pieces += [v[:, h * DH:(h + 1) * DH], ones, zer]
    vext_ref[...] = jnp.concatenate(pieces, axis=1)
    i0 = jax.lax.broadcasted_iota(jnp.int32, (D, H), 0) // DH
    i1 = jax.lax.broadcasted_iota(jnp.int32, (D, H), 1)
    sel = (i0 == i1).astype(jnp.float32)
    nq = jnp.sqrt(jnp.dot(q * q, sel, preferred_element_type=jnp.float32))
    nk2 = jnp.dot(k * k, sel, preferred_element_type=jnp.float32)
    kmax = jnp.sqrt(jnp.max(nk2, axis=0, keepdims=True))
    bbs = []
    for h in range(H):
        s = jnp.abs(wpair_ref[NRBF, h])
        for c in range(NRBF):
            s += jnp.abs(wpair_ref[c, h])
        bbs.append(s.reshape(1, 1))
    bb = jnp.concatenate(bbs, axis=1)
    mb_ref[...] = 0.25 * nq * kmax + bb


def _attn_body(a_ref, b_ref, ci_ref, cj_ref, q_ref, k_ref, vext_ref, mb_ref,
               wpair_ref, acc_ref):
    jb = pl.program_id(1)

    @pl.when(jb == 0)
    def _reset():
        acc_ref[...] = jnp.zeros((BLK, 2 * D), dtype=jnp.float32)

    d2 = jax.lax.dot_general(a_ref[...], b_ref[...], (((1,), (1,)), ((), ())),
                             preferred_element_type=jnp.float32)
    d2 = jnp.maximum(d2, 0.0) + 1e-8
    d = jnp.sqrt(d2)
    t = d2 * -8.0
    same = (ci_ref[...] == cj_ref[0:1, :]).astype(jnp.bfloat16)
    rs = []
    for c in range(NRBF):
        cc = 2.0 * c / (NRBF - 1)
        rs.append(jnp.exp(t + (16.0 * cc) * d - (8.0 * cc * cc)).astype(jnp.bfloat16))

    q = q_ref[...]
    k = k_ref[...]
    vt = vext_ref[...]
    mb = mb_ref[...]
    scale = 1.0 / (DH ** 0.5)
    for h in range(H):
        sl = slice(h * DH, (h + 1) * DH)
        logits = jax.lax.dot_general(q[:, sl] * scale, k[:, sl],
                                     (((1,), (1,)), ((), ())),
                                     preferred_element_type=jnp.float32)
        bias = same * wpair_ref[NRBF, h].astype(jnp.bfloat16)
        for c in range(NRBF):
            bias += rs[c] * wpair_ref[c, h].astype(jnp.bfloat16)
        arg = (logits - mb[:, h:h + 1]) + bias.astype(jnp.float32)
        p = jnp.exp(jnp.maximum(arg, -80.0))
        sl2 = slice(h * 2 * DH, (h + 1) * 2 * DH)
        acc_ref[:, sl2] += jnp.dot(p, vt[:, sl2],
                                   preferred_element_type=jnp.float32)


def _ff_body(acc_ref, h_ref, wo_ref, w1_ref, w2_ref, out_ref):
    acc = acc_ref[...]
    o = jnp.concatenate(
        [acc[:, 2 * h * DH:(2 * h + 1) * DH]
         / acc[:, (2 * h + 1) * DH:(2 * h + 1) * DH + 1] for h in range(H)],
        axis=1)
    h1 = h_ref[...] + jnp.dot(o, wo_ref[...], preferred_element_type=jnp.float32)
    ffin = jax.nn.relu(jnp.dot(_ln(h1), w1_ref[...], preferred_element_type=jnp.float32))
    out_ref[...] = h1 + jnp.dot(ffin, w2_ref[...], preferred_element_type=jnp.float32)


def _attn_call(a_mat, b_mat, ci, cj8, q, k, vext, mb, wpair_l):
    return pl.pallas_call(
        _attn_body,
        grid=(IB, JB),
        in_specs=[
            pl.BlockSpec((BLK, 8), lambda i, j: (i, 0)),
            pl.BlockSpec((BLK, 8), lambda i, j: (j, 0)),
            pl.BlockSpec((BLK, 1), lambda i, j: (i, 0)),
            pl.BlockSpec((8, BLK), lambda i, j: (0, j)),
            pl.BlockSpec((BLK, D), lambda i, j: (i, 0)),
            pl.BlockSpec((BLK, D), lambda i, j: (j, 0)),
            pl.BlockSpec((BLK, 2 * D), lambda i, j: (j, 0)),
            pl.BlockSpec((BLK, H), lambda i, j: (i, 0)),
            pl.BlockSpec(memory_space=pltpu.SMEM),
        ],
        out_specs=pl.BlockSpec((BLK, 2 * D), lambda i, j: (i, 0)),
        out_shape=jax.ShapeDtypeStruct((N, 2 * D), jnp.float32),
        compiler_params=pltpu.CompilerParams(
            dimension_semantics=("arbitrary", "arbitrary")),
    )(a_mat, b_mat, ci, cj8, q, k, vext, mb, wpair_l)


def kernel(x, edge_index, edge_attr, aa_embed, w_pos, b_pos, wq, wk, wv, wo,
           w_pair, w1, w2):
    apx = x[:, 0:42:3]
    apy = x[:, 1:42:3]
    apz = x[:, 2:42:3]
    mask = x[:, 45:59]
    aa_col = x[:, 42:43]
    chain = x[:, 44].astype(jnp.int32)
    ci = chain.reshape(N, 1)
    cj8 = jnp.broadcast_to(chain.reshape(1, N), (8, N))
    wx = w_pos[0::3]
    wy = w_pos[1::3]
    wz = w_pos[2::3]
    bpos = b_pos.reshape(1, D)

    h0, a_mat, b_mat = pl.pallas_call(
        _prep_body,
        out_shape=[
            jax.ShapeDtypeStruct((N, D), jnp.float32),
            jax.ShapeDtypeStruct((N, 8), jnp.float32),
            jax.ShapeDtypeStruct((N, 8), jnp.float32),
        ],
    )(apx, apy, apz, mask, aa_col, wx, wy, wz, aa_embed, bpos)

    pre_call = pl.pallas_call(
        _pre_body,
        in_specs=[
            pl.BlockSpec((N, D), lambda: (0, 0)),
            pl.BlockSpec((D, D), lambda: (0, 0)),
            pl.BlockSpec((D, D), lambda: (0, 0)),
            pl.BlockSpec((D, D), lambda: (0, 0)),
            pl.BlockSpec(memory_space=pltpu.SMEM),
        ],
        out_specs=[
            pl.BlockSpec((N, D), lambda: (0, 0)),
            pl.BlockSpec((N, D), lambda: (0, 0)),
            pl.BlockSpec((N, 2 * D), lambda: (0, 0)),
            pl.BlockSpec((N, H), lambda: (0, 0)),
        ],
        out_shape=[
            jax.ShapeDtypeStruct((N, D), jnp.float32),
            jax.ShapeDtypeStruct((N, D), jnp.float32),
            jax.ShapeDtypeStruct((N, 2 * D), jnp.float32),
            jax.ShapeDtypeStruct((N, H), jnp.float32),
        ],
    )

    ff_call = pl.pallas_call(
        _ff_body,
        out_shape=jax.ShapeDtypeStruct((N, D), jnp.float32),
    )

    h = h0
    for l in range(L):
        q, k, vext, mb = pre_call(h, wq[l], wk[l], wv[l], w_pair[l])
        acc = _attn_call(a_mat, b_mat, ci, cj8, q, k, vext, mb, w_pair[l])
        h = ff_call(acc, h, wo[l], w1[l], w2[l])
    return h.reshape(1, N, D)


# speedup vs baseline: 3.3502x; 1.1279x over previous
import math

import jax
import jax.numpy as jnp
from jax.experimental import pallas as pl
from jax.experimental.pallas import tpu as pltpu

N = 2048
D = 128
H = 8
DH = D // H
L = 2
FF = 256
NRBF = 8
NAA = 21
BLK = 256
IB = N // BLK
JB = N // BLK
EHW = 2 * DH

_EXPC = [math.exp(-8.0 * (2.0 * c / (NRBF - 1)) ** 2) for c in range(NRBF)]


def _ln(h):
    mu = jnp.mean(h, axis=-1, keepdims=True)
    hc = h - mu
    var = jnp.mean(hc * hc, axis=-1, keepdims=True)
    return hc / jnp.sqrt(var + 1e-5)


def _prep_body(apx_ref, apy_ref, apz_ref, mask_ref, aa_ref, ch_ref, wx_ref,
               wy_ref, wz_ref, emb_ref, bpos_ref, h0_ref, a_ref, b_ref):
    m = (mask_ref[...] > 0.5).astype(jnp.float32)
    apx = apx_ref[...]
    apy = apy_ref[...]
    apz = apz_ref[...]
    cax = apx[:, 1:2]
    cay = apy[:, 1:2]
    caz = apz[:, 1:2]
    h = jnp.dot((apx - cax) * m, wx_ref[...], preferred_element_type=jnp.float32)
    h += jnp.dot((apy - cay) * m, wy_ref[...], preferred_element_type=jnp.float32)
    h += jnp.dot((apz - caz) * m, wz_ref[...], preferred_element_type=jnp.float32)
    ai = jnp.clip(aa_ref[...].astype(jnp.int32), 0, NAA - 1)
    iota = jax.lax.broadcasted_iota(jnp.int32, (N, NAA), 1)
    onehot = (ai == iota).astype(jnp.float32)
    h += jnp.dot(onehot, emb_ref[...], preferred_element_type=jnp.float32)
    h0_ref[...] = h + bpos_ref[...]
    ch = ch_ref[...]
    n2 = cax * cax + cay * cay + caz * caz
    c2 = ch * ch
    ones = jnp.ones_like(n2)
    zeros = jnp.zeros_like(n2)
    a_ref[...] = jnp.concatenate(
        [-2.0 * cax, -2.0 * cay, -2.0 * caz, n2, ones,
         c2, -2.0 * ch, ones,
         zeros, zeros, zeros, zeros, zeros, zeros, zeros, zeros], axis=1)
    b_ref[...] = jnp.concatenate(
        [cax, cay, caz, ones, n2,
         ones, ch, c2,
         zeros, zeros, zeros, zeros, zeros, zeros, zeros, zeros], axis=1)


def _pre_body(h_ref, wq_ref, wk_ref, wv_ref, wpair_ref,
              qext_ref, kext_ref, vext_ref):
    hn = _ln(h_ref[...])
    q = jnp.dot(hn, wq_ref[...], preferred_element_type=jnp.float32)
    k = jnp.dot(hn, wk_ref[...], preferred_element_type=jnp.float32)
    v = jnp.dot(hn, wv_ref[...], preferred_element_type=jnp.float32)
    ones = jnp.ones((N, 1), dtype=jnp.float32)
    zer = jnp.zeros((N, DH - 1), dtype=jnp.float32)
    i0 = jax.lax.broadcasted_iota(jnp.int32, (D, H), 0) // DH
    i1 = jax.lax.broadcasted_iota(jnp.int32, (D, H), 1)
    sel = (i0 == i1).astype(jnp.float32)
    nq = jnp.sqrt(jnp.dot(q * q, sel, preferred_element_type=jnp.float32))
    nk2 = jnp.dot(k * k, sel, preferred_element_type=jnp.float32)
    kmax = jnp.sqrt(jnp.max(nk2, axis=0, keepdims=True))
    bbs = []
    for h in range(H):
        s = jnp.abs(wpair_ref[NRBF, h])
        for c in range(NRBF):
            s += jnp.abs(wpair_ref[c, h])
        bbs.append(s.reshape(1, 1))
    bb = jnp.concatenate(bbs, axis=1)
    mb = 0.25 * nq * kmax + bb
    qp, kp, vp = [], [], []
    for h in range(H):
        sl = slice(h * DH, (h + 1) * DH)
        qp += [q[:, sl] * 0.25, mb[:, h:h + 1], zer]
        kp += [k[:, sl], -ones, zer]
        vp += [v[:, sl], ones, zer]
    qext_ref[...] = jnp.concatenate(qp, axis=1)
    kext_ref[...] = jnp.concatenate(kp, axis=1)
    vext_ref[...] = jnp.concatenate(vp, axis=1)


def _attn_body(a_ref, b_ref, qext_ref, kext_ref, vext_ref, wpair_ref, acc_ref):
    jb = pl.program_id(1)

    @pl.when(jb == 0)
    def _reset():
        acc_ref[...] = jnp.zeros((BLK, H * EHW), dtype=jnp.float32)

    a = a_ref[...]
    b = b_ref[...]
    d2 = jax.lax.dot_general(a[:, 0:5], b[:, 0:5], (((1,), (1,)), ((), ())),
                             preferred_element_type=jnp.float32)
    dc2 = jax.lax.dot_general(a[:, 5:8], b[:, 5:8], (((1,), (1,)), ((), ())),
                              preferred_element_type=jnp.float32)
    d2 = jnp.maximum(d2, 0.0) + 1e-8
    d = jnp.sqrt(d2)
    same = jnp.exp(dc2 * -30.0).astype(jnp.bfloat16)
    e0 = jnp.exp(d2 * -8.0).astype(jnp.bfloat16)
    u1 = jnp.exp(d * (32.0 / 7.0))
    u2 = u1 * u1
    u3 = u2 * u1
    u4 = u2 * u2
    u5 = u4 * u1
    u6 = u4 * u2
    u7 = u4 * u3
    ub = [None] + [x.astype(jnp.bfloat16) for x in (u1, u2, u3, u4, u5, u6, u7)]

    qx = qext_ref[...]
    kx = kext_ref[...]
    vt = vext_ref[...]
    for h in range(H):
        sl = slice(h * EHW, (h + 1) * EHW)
        logits = jax.lax.dot_general(qx[:, sl], kx[:, sl],
                                     (((1,), (1,)), ((), ())),
                                     preferred_element_type=jnp.float32)
        s = None
        for c in range(1, NRBF):
            ac = (wpair_ref[c, h] * _EXPC[c]).astype(jnp.bfloat16)
            s = ub[c] * ac if s is None else s + ub[c] * ac
        s += wpair_ref[0, h].astype(jnp.bfloat16)
        bias = s * e0 + same * wpair_ref[NRBF, h].astype(jnp.bfloat16)
        arg = logits + bias.astype(jnp.float32)
        p = jnp.exp(jnp.maximum(arg, -80.0))
        acc_ref[:, sl] += jnp.dot(p, vt[:, sl],
                                  preferred_element_type=jnp.float32)


def _ff_body(acc_ref, h_ref, wo_ref, w1_ref, w2_ref, out_ref):
    acc = acc_ref[...]
    o = jnp.concatenate(
        [acc[:, h * EHW:h * EHW + DH]
         / acc[:, h * EHW + DH:h * EHW + DH + 1] for h in range(H)],
        axis=1)
    h1 = h_ref[...] + jnp.dot(o, wo_ref[...], preferred_element_type=jnp.float32)
    ffin = jax.nn.relu(jnp.dot(_ln(h1), w1_ref[...], preferred_element_type=jnp.float32))
    out_ref[...] = h1 + jnp.dot(ffin, w2_ref[...], preferred_element_type=jnp.float32)


def _attn_call(a_mat, b_mat, qext, kext, vext, wpair_l):
    return pl.pallas_call(
        _attn_body,
        grid=(IB, JB),
        in_specs=[
            pl.BlockSpec((BLK, 16), lambda i, j: (i, 0)),
            pl.BlockSpec((BLK, 16), lambda i, j: (j, 0)),
            pl.BlockSpec((BLK, H * EHW), lambda i, j: (i, 0)),
            pl.BlockSpec((BLK, H * EHW), lambda i, j: (j, 0)),
            pl.BlockSpec((BLK, H * EHW), lambda i, j: (j, 0)),
            pl.BlockSpec(memory_space=pltpu.SMEM),
        ],
        out_specs=pl.BlockSpec((BLK, H * EHW), lambda i, j: (i, 0)),
        out_shape=jax.ShapeDtypeStruct((N, H * EHW), jnp.float32),
        compiler_params=pltpu.CompilerParams(
            dimension_semantics=("arbitrary", "arbitrary")),
    )(a_mat, b_mat, qext, kext, vext, wpair_l)


def kernel(x, edge_index, edge_attr, aa_embed, w_pos, b_pos, wq, wk, wv, wo,
           w_pair, w1, w2):
    apx = x[:, 0:42:3]
    apy = x[:, 1:42:3]
    apz = x[:, 2:42:3]
    mask = x[:, 45:59]
    aa_col = x[:, 42:43]
    chf = x[:, 44:45].astype(jnp.int32).astype(jnp.float32)
    wx = w_pos[0::3]
    wy = w_pos[1::3]
    wz = w_pos[2::3]
    bpos = b_pos.reshape(1, D)

    h0, a_mat, b_mat = pl.pallas_call(
        _prep_body,
        out_shape=[
            jax.ShapeDtypeStruct((N, D), jnp.float32),
            jax.ShapeDtypeStruct((N, 16), jnp.float32),
            jax.ShapeDtypeStruct((N, 16), jnp.float32),
        ],
    )(apx, apy, apz, mask, aa_col, chf, wx, wy, wz, aa_embed, bpos)

    pre_call = pl.pallas_call(
        _pre_body,
        in_specs=[
            pl.BlockSpec((N, D), lambda: (0, 0)),
            pl.BlockSpec((D, D), lambda: (0, 0)),
            pl.BlockSpec((D, D), lambda: (0, 0)),
            pl.BlockSpec((D, D), lambda: (0, 0)),
            pl.BlockSpec(memory_space=pltpu.SMEM),
        ],
        out_specs=[
            pl.BlockSpec((N, H * EHW), lambda: (0, 0)),
            pl.BlockSpec((N, H * EHW), lambda: (0, 0)),
            pl.BlockSpec((N, H * EHW), lambda: (0, 0)),
        ],
        out_shape=[
            jax.ShapeDtypeStruct((N, H * EHW), jnp.float32),
            jax.ShapeDtypeStruct((N, H * EHW), jnp.float32),
            jax.ShapeDtypeStruct((N, H * EHW), jnp.float32),
        ],
    )

    ff_call = pl.pallas_call(
        _ff_body,
        out_shape=jax.ShapeDtypeStruct((N, D), jnp.float32),
    )

    h = h0
    for l in range(L):
        qext, kext, vext = pre_call(h, wq[l], wk[l], wv[l], w_pair[l])
        acc = _attn_call(a_mat, b_mat, qext, kext, vext, w_pair[l])
        h = ff_call(acc, h, wo[l], w1[l], w2[l])
    return h.reshape(1, N, D)


# speedup vs baseline: 3.6985x; 1.1040x over previous
import math

import jax
import jax.numpy as jnp
from jax.experimental import pallas as pl
from jax.experimental.pallas import tpu as pltpu

N = 2048
D = 128
H = 8
DH = D // H
L = 2
FF = 256
NRBF = 8
NAA = 21
BLK = 256
IB = N // BLK
JB = N // BLK
EHW = 2 * DH

_EXPC = [math.exp(-8.0 * (2.0 * c / (NRBF - 1)) ** 2) for c in range(NRBF)]


def _ln(h):
    mu = jnp.mean(h, axis=-1, keepdims=True)
    hc = h - mu
    var = jnp.mean(hc * hc, axis=-1, keepdims=True)
    return hc / jnp.sqrt(var + 1e-5)


def _prep_body(apx_ref, apy_ref, apz_ref, mask_ref, aa_ref, ch_ref, wx_ref,
               wy_ref, wz_ref, emb_ref, bpos_ref, h0_ref, a_ref, b_ref):
    m = (mask_ref[...] > 0.5).astype(jnp.float32)
    apx = apx_ref[...]
    apy = apy_ref[...]
    apz = apz_ref[...]
    cax = apx[:, 1:2]
    cay = apy[:, 1:2]
    caz = apz[:, 1:2]
    h = jnp.dot((apx - cax) * m, wx_ref[...], preferred_element_type=jnp.float32)
    h += jnp.dot((apy - cay) * m, wy_ref[...], preferred_element_type=jnp.float32)
    h += jnp.dot((apz - caz) * m, wz_ref[...], preferred_element_type=jnp.float32)
    ai = jnp.clip(aa_ref[...].astype(jnp.int32), 0, NAA - 1)
    iota = jax.lax.broadcasted_iota(jnp.int32, (N, NAA), 1)
    onehot = (ai == iota).astype(jnp.float32)
    h += jnp.dot(onehot, emb_ref[...], preferred_element_type=jnp.float32)
    h0_ref[...] = h + bpos_ref[...]
    ch = ch_ref[...]
    n2 = cax * cax + cay * cay + caz * caz
    c2 = ch * ch
    ones = jnp.ones_like(n2)
    zeros = jnp.zeros_like(n2)
    a_ref[...] = jnp.concatenate(
        [-2.0 * cax, -2.0 * cay, -2.0 * caz, n2, ones,
         c2, -2.0 * ch, ones,
         zeros, zeros, zeros, zeros, zeros, zeros, zeros, zeros], axis=1)
    b_ref[...] = jnp.concatenate(
        [cax, cay, caz, ones, n2,
         ones, ch, c2,
         zeros, zeros, zeros, zeros, zeros, zeros, zeros, zeros], axis=1)


def _pre_body(h_ref, wq_ref, wk_ref, wv_ref, wpair_ref,
              qext_ref, kext_ref, vext_ref):
    hn = _ln(h_ref[...])
    q = jnp.dot(hn, wq_ref[...], preferred_element_type=jnp.float32)
    k = jnp.dot(hn, wk_ref[...], preferred_element_type=jnp.float32)
    v = jnp.dot(hn, wv_ref[...], preferred_element_type=jnp.float32)
    i0 = jax.lax.broadcasted_iota(jnp.int32, (D, H), 0) // DH
    i1 = jax.lax.broadcasted_iota(jnp.int32, (D, H), 1)
    sel = (i0 == i1).astype(jnp.float32)
    nq = jnp.sqrt(jnp.dot(q * q, sel, preferred_element_type=jnp.float32))
    nk2 = jnp.dot(k * k, sel, preferred_element_type=jnp.float32)
    kmax = jnp.sqrt(jnp.max(nk2, axis=0, keepdims=True))
    bbs = []
    for h in range(H):
        s = jnp.abs(wpair_ref[NRBF, h])
        for c in range(NRBF):
            s += jnp.abs(wpair_ref[c, h])
        bbs.append(s.reshape(1, 1))
    bb = jnp.concatenate(bbs, axis=1)
    mb = 0.25 * nq * kmax + bb
    j0 = jax.lax.broadcasted_iota(jnp.int32, (D, H * EHW), 0)
    j1 = jax.lax.broadcasted_iota(jnp.int32, (D, H * EHW), 1)
    head = j1 // EHW
    lane = j1 % EHW
    p1 = ((lane < DH) & (j0 == head * DH + lane)).astype(jnp.float32)
    m0 = jax.lax.broadcasted_iota(jnp.int32, (H, H * EHW), 0)
    m1 = jax.lax.broadcasted_iota(jnp.int32, (H, H * EHW), 1)
    pm = ((m1 % EHW == DH) & (m0 == m1 // EHW)).astype(jnp.float32)
    r1 = jax.lax.broadcasted_iota(jnp.int32, (1, H * EHW), 1)
    is_flag = (r1 % EHW == DH)
    negrow = jnp.where(is_flag, -1.0, 0.0)
    qext_ref[...] = (jnp.dot(q, p1 * 0.25, preferred_element_type=jnp.float32)
                     + jnp.dot(mb, pm, preferred_element_type=jnp.float32))
    kext_ref[...] = jnp.dot(k, p1, preferred_element_type=jnp.float32) + negrow
    vext_ref[...] = jnp.dot(v, p1, preferred_element_type=jnp.float32) - negrow


def _attn_body(a_ref, b_ref, qext_ref, kext_ref, vext_ref, wpair_ref, acc_ref):
    jb = pl.program_id(1)

    @pl.when(jb == 0)
    def _reset():
        acc_ref[...] = jnp.zeros((BLK, H * EHW), dtype=jnp.float32)

    a = a_ref[...]
    b = b_ref[...]
    d2 = jax.lax.dot_general(a[:, 0:5], b[:, 0:5], (((1,), (1,)), ((), ())),
                             preferred_element_type=jnp.float32)
    dc2 = jax.lax.dot_general(a[:, 5:8], b[:, 5:8], (((1,), (1,)), ((), ())),
                              preferred_element_type=jnp.float32)
    d2 = jnp.maximum(d2, 0.0) + 1e-8
    d = jnp.sqrt(d2)
    same = jnp.exp(dc2 * -30.0).astype(jnp.bfloat16)
    e0 = jnp.exp(d2 * -8.0).astype(jnp.bfloat16)
    u1 = jnp.exp(d * (32.0 / 7.0))
    u2 = u1 * u1
    u3 = u2 * u1
    u4 = u2 * u2
    u5 = u4 * u1
    u6 = u4 * u2
    u7 = u4 * u3
    ub = [None] + [x.astype(jnp.bfloat16) for x in (u1, u2, u3, u4, u5, u6, u7)]

    qx = qext_ref[...]
    kx = kext_ref[...]
    vt = vext_ref[...]
    for h in range(H):
        sl = slice(h * EHW, (h + 1) * EHW)
        logits = jax.lax.dot_general(qx[:, sl], kx[:, sl],
                                     (((1,), (1,)), ((), ())),
                                     preferred_element_type=jnp.float32)
        s = None
        for c in range(1, NRBF):
            ac = (wpair_ref[c, h] * _EXPC[c]).astype(jnp.bfloat16)
            s = ub[c] * ac if s is None else s + ub[c] * ac
        s += wpair_ref[0, h].astype(jnp.bfloat16)
        bias = s * e0 + same * wpair_ref[NRBF, h].astype(jnp.bfloat16)
        arg = logits + bias.astype(jnp.float32)
        p = jnp.exp(jnp.maximum(arg, -80.0))
        acc_ref[:, sl] += jnp.dot(p, vt[:, sl],
                                  preferred_element_type=jnp.float32)


def _ff_body(acc_ref, h_ref, wo_ref, w1_ref, w2_ref, out_ref):
    acc = acc_ref[...]
    j0 = jax.lax.broadcasted_iota(jnp.int32, (H * EHW, D), 0)
    j1 = jax.lax.broadcasted_iota(jnp.int32, (H * EHW, D), 1)
    head = j1 // DH
    pn = (j0 == head * EHW + (j1 % DH)).astype(jnp.float32)
    pd = (j0 == head * EHW + DH).astype(jnp.float32)
    num = jnp.dot(acc, pn, preferred_element_type=jnp.float32)
    den = jnp.dot(acc, pd, preferred_element_type=jnp.float32)
    o = num / den
    h1 = h_ref[...] + jnp.dot(o, wo_ref[...], preferred_element_type=jnp.float32)
    ffin = jax.nn.relu(jnp.dot(_ln(h1), w1_ref[...], preferred_element_type=jnp.float32))
    out_ref[...] = h1 + jnp.dot(ffin, w2_ref[...], preferred_element_type=jnp.float32)


def _attn_call(a_mat, b_mat, qext, kext, vext, wpair_l):
    return pl.pallas_call(
        _attn_body,
        grid=(IB, JB),
        in_specs=[
            pl.BlockSpec((BLK, 16), lambda i, j: (i, 0)),
            pl.BlockSpec((BLK, 16), lambda i, j: (j, 0)),
            pl.BlockSpec((BLK, H * EHW), lambda i, j: (i, 0)),
            pl.BlockSpec((BLK, H * EHW), lambda i, j: (j, 0)),
            pl.BlockSpec((BLK, H * EHW), lambda i, j: (j, 0)),
            pl.BlockSpec(memory_space=pltpu.SMEM),
        ],
        out_specs=pl.BlockSpec((BLK, H * EHW), lambda i, j: (i, 0)),
        out_shape=jax.ShapeDtypeStruct((N, H * EHW), jnp.float32),
        compiler_params=pltpu.CompilerParams(
            dimension_semantics=("arbitrary", "arbitrary")),
    )(a_mat, b_mat, qext, kext, vext, wpair_l)


def kernel(x, edge_index, edge_attr, aa_embed, w_pos, b_pos, wq, wk, wv, wo,
           w_pair, w1, w2):
    apx = x[:, 0:42:3]
    apy = x[:, 1:42:3]
    apz = x[:, 2:42:3]
    mask = x[:, 45:59]
    aa_col = x[:, 42:43]
    chf = x[:, 44:45].astype(jnp.int32).astype(jnp.float32)
    wx = w_pos[0::3]
    wy = w_pos[1::3]
    wz = w_pos[2::3]
    bpos = b_pos.reshape(1, D)

    h0, a_mat, b_mat = pl.pallas_call(
        _prep_body,
        out_shape=[
            jax.ShapeDtypeStruct((N, D), jnp.float32),
            jax.ShapeDtypeStruct((N, 16), jnp.float32),
            jax.ShapeDtypeStruct((N, 16), jnp.float32),
        ],
    )(apx, apy, apz, mask, aa_col, chf, wx, wy, wz, aa_embed, bpos)

    pre_call = pl.pallas_call(
        _pre_body,
        in_specs=[
            pl.BlockSpec((N, D), lambda: (0, 0)),
            pl.BlockSpec((D, D), lambda: (0, 0)),
            pl.BlockSpec((D, D), lambda: (0, 0)),
            pl.BlockSpec((D, D), lambda: (0, 0)),
            pl.BlockSpec(memory_space=pltpu.SMEM),
        ],
        out_specs=[
            pl.BlockSpec((N, H * EHW), lambda: (0, 0)),
            pl.BlockSpec((N, H * EHW), lambda: (0, 0)),
            pl.BlockSpec((N, H * EHW), lambda: (0, 0)),
        ],
        out_shape=[
            jax.ShapeDtypeStruct((N, H * EHW), jnp.float32),
            jax.ShapeDtypeStruct((N, H * EHW), jnp.float32),
            jax.ShapeDtypeStruct((N, H * EHW), jnp.float32),
        ],
    )

    ff_call = pl.pallas_call(
        _ff_body,
        out_shape=jax.ShapeDtypeStruct((N, D), jnp.float32),
    )

    h = h0
    for l in range(L):
        qext, kext, vext = pre_call(h, wq[l], wk[l], wv[l], w_pair[l])
        acc = _attn_call(a_mat, b_mat, qext, kext, vext, w_pair[l])
        h = ff_call(acc, h, wo[l], w1[l], w2[l])
    return h.reshape(1, N, D)


# speedup vs baseline: 4.1437x; 1.1204x over previous
import math

import jax
import jax.numpy as jnp
from jax.experimental import pallas as pl
from jax.experimental.pallas import tpu as pltpu

N = 2048
D = 128
H = 8
DH = D // H
L = 2
FF = 256
NRBF = 8
NAA = 21
BLK = 256
BLKJ = 512
IB = N // BLK
JB = N // BLKJ
EHW = 2 * DH

_EXPC = [math.exp(-8.0 * (2.0 * c / (NRBF - 1)) ** 2) for c in range(NRBF)]


def _ln(h):
    mu = jnp.mean(h, axis=-1, keepdims=True)
    hc = h - mu
    var = jnp.mean(hc * hc, axis=-1, keepdims=True)
    return hc / jnp.sqrt(var + 1e-5)


def _prep_body(apx_ref, apy_ref, apz_ref, mask_ref, aa_ref, ch_ref, wx_ref,
               wy_ref, wz_ref, emb_ref, bpos_ref, h0_ref, a_ref, b_ref):
    m = (mask_ref[...] > 0.5).astype(jnp.float32)
    apx = apx_ref[...]
    apy = apy_ref[...]
    apz = apz_ref[...]
    cax = apx[:, 1:2]
    cay = apy[:, 1:2]
    caz = apz[:, 1:2]
    h = jnp.dot((apx - cax) * m, wx_ref[...], preferred_element_type=jnp.float32)
    h += jnp.dot((apy - cay) * m, wy_ref[...], preferred_element_type=jnp.float32)
    h += jnp.dot((apz - caz) * m, wz_ref[...], preferred_element_type=jnp.float32)
    ai = jnp.clip(aa_ref[...].astype(jnp.int32), 0, NAA - 1)
    iota = jax.lax.broadcasted_iota(jnp.int32, (N, NAA), 1)
    onehot = (ai == iota).astype(jnp.float32)
    h += jnp.dot(onehot, emb_ref[...], preferred_element_type=jnp.float32)
    h0_ref[...] = h + bpos_ref[...]
    ch = ch_ref[...]
    n2 = cax * cax + cay * cay + caz * caz
    c2 = ch * ch
    ones = jnp.ones_like(n2)
    zeros = jnp.zeros_like(n2)
    a_ref[...] = jnp.concatenate(
        [-2.0 * cax, -2.0 * cay, -2.0 * caz, n2, ones,
         c2, -2.0 * ch, ones,
         zeros, zeros, zeros, zeros, zeros, zeros, zeros, zeros], axis=1)
    b_ref[...] = jnp.concatenate(
        [cax, cay, caz, ones, n2,
         ones, ch, c2,
         zeros, zeros, zeros, zeros, zeros, zeros, zeros, zeros], axis=1)


def _pre_body(h_ref, wq_ref, wk_ref, wv_ref, wpair_ref,
              qext_ref, kext_ref, vext_ref):
    hn = _ln(h_ref[...])
    q = jnp.dot(hn, wq_ref[...], preferred_element_type=jnp.float32)
    k = jnp.dot(hn, wk_ref[...], preferred_element_type=jnp.float32)
    v = jnp.dot(hn, wv_ref[...], preferred_element_type=jnp.float32)
    i0 = jax.lax.broadcasted_iota(jnp.int32, (D, H), 0) // DH
    i1 = jax.lax.broadcasted_iota(jnp.int32, (D, H), 1)
    sel = (i0 == i1).astype(jnp.float32)
    nq = jnp.sqrt(jnp.dot(q * q, sel, preferred_element_type=jnp.float32))
    nk2 = jnp.dot(k * k, sel, preferred_element_type=jnp.float32)
    kmax = jnp.sqrt(jnp.max(nk2, axis=0, keepdims=True))
    bbs = []
    for h in range(H):
        s = jnp.abs(wpair_ref[NRBF, h])
        for c in range(NRBF):
            s += jnp.abs(wpair_ref[c, h])
        bbs.append(s.reshape(1, 1))
    bb = jnp.concatenate(bbs, axis=1)
    mb = 0.25 * nq * kmax + bb
    j0 = jax.lax.broadcasted_iota(jnp.int32, (D, H * EHW), 0)
    j1 = jax.lax.broadcasted_iota(jnp.int32, (D, H * EHW), 1)
    head = j1 // EHW
    lane = j1 % EHW
    p1 = ((lane < DH) & (j0 == head * DH + lane)).astype(jnp.float32)
    m0 = jax.lax.broadcasted_iota(jnp.int32, (H, H * EHW), 0)
    m1 = jax.lax.broadcasted_iota(jnp.int32, (H, H * EHW), 1)
    pm = ((m1 % EHW == DH) & (m0 == m1 // EHW)).astype(jnp.float32)
    r1 = jax.lax.broadcasted_iota(jnp.int32, (1, H * EHW), 1)
    is_flag = (r1 % EHW == DH)
    negrow = jnp.where(is_flag, -1.0, 0.0)
    qext_ref[...] = (jnp.dot(q, p1 * 0.25, preferred_element_type=jnp.float32)
                     + jnp.dot(mb, pm, preferred_element_type=jnp.float32))
    kext_ref[...] = jnp.dot(k, p1, preferred_element_type=jnp.float32) + negrow
    vext_ref[...] = jnp.dot(v, p1, preferred_element_type=jnp.float32) - negrow


def _attn_body(a_ref, b_ref, qext_ref, kext_ref, vext_ref, wpair_ref, acc_ref):
    jb = pl.program_id(1)

    @pl.when(jb == 0)
    def _reset():
        acc_ref[...] = jnp.zeros((BLK, H * EHW), dtype=jnp.float32)

    a = a_ref[...]
    b = b_ref[...]
    d2 = jax.lax.dot_general(a[:, 0:5], b[:, 0:5], (((1,), (1,)), ((), ())),
                             preferred_element_type=jnp.float32)
    dc2 = jax.lax.dot_general(a[:, 5:8], b[:, 5:8], (((1,), (1,)), ((), ())),
                              preferred_element_type=jnp.float32)
    d2 = jnp.maximum(d2, 0.0) + 1e-8
    d = jnp.sqrt(d2)
    same = jnp.exp(dc2 * -30.0).astype(jnp.bfloat16)
    e0 = jnp.exp(d2 * -8.0).astype(jnp.bfloat16)
    u1 = jnp.exp(d * (32.0 / 7.0))
    u2 = u1 * u1
    u3 = u2 * u1
    u4 = u2 * u2
    u5 = u4 * u1
    u6 = u4 * u2
    u7 = u4 * u3
    ub = [None] + [x.astype(jnp.bfloat16) for x in (u1, u2, u3, u4, u5, u6, u7)]

    qx = qext_ref[...]
    kx = kext_ref[...]
    vt = vext_ref[...]
    for h in range(H):
        sl = slice(h * EHW, (h + 1) * EHW)
        logits = jax.lax.dot_general(qx[:, sl], kx[:, sl],
                                     (((1,), (1,)), ((), ())),
                                     preferred_element_type=jnp.float32)
        s = None
        for c in range(1, NRBF):
            ac = (wpair_ref[c, h] * _EXPC[c]).astype(jnp.bfloat16)
            s = ub[c] * ac if s is None else s + ub[c] * ac
        s += wpair_ref[0, h].astype(jnp.bfloat16)
        bias = s * e0 + same * wpair_ref[NRBF, h].astype(jnp.bfloat16)
        arg = logits + bias.astype(jnp.float32)
        p = jnp.exp(jnp.maximum(arg, -80.0))
        acc_ref[:, sl] += jnp.dot(p, vt[:, sl],
                                  preferred_element_type=jnp.float32)


def _ff_body(acc_ref, h_ref, wo_ref, w1_ref, w2_ref, out_ref):
    acc = acc_ref[...]
    j0 = jax.lax.broadcasted_iota(jnp.int32, (H * EHW, D), 0)
    j1 = jax.lax.broadcasted_iota(jnp.int32, (H * EHW, D), 1)
    head = j1 // DH
    pn = (j0 == head * EHW + (j1 % DH)).astype(jnp.float32)
    pd = (j0 == head * EHW + DH).astype(jnp.float32)
    num = jnp.dot(acc, pn, preferred_element_type=jnp.float32)
    den = jnp.dot(acc, pd, preferred_element_type=jnp.float32)
    o = num / den
    h1 = h_ref[...] + jnp.dot(o, wo_ref[...], preferred_element_type=jnp.float32)
    ffin = jax.nn.relu(jnp.dot(_ln(h1), w1_ref[...], preferred_element_type=jnp.float32))
    out_ref[...] = h1 + jnp.dot(ffin, w2_ref[...], preferred_element_type=jnp.float32)


def _attn_call(a_mat, b_mat, qext, kext, vext, wpair_l):
    return pl.pallas_call(
        _attn_body,
        grid=(IB, JB),
        in_specs=[
            pl.BlockSpec((BLK, 16), lambda i, j: (i, 0)),
            pl.BlockSpec((BLKJ, 16), lambda i, j: (j, 0)),
            pl.BlockSpec((BLK, H * EHW), lambda i, j: (i, 0)),
            pl.BlockSpec((BLKJ, H * EHW), lambda i, j: (j, 0)),
            pl.BlockSpec((BLKJ, H * EHW), lambda i, j: (j, 0)),
            pl.BlockSpec(memory_space=pltpu.SMEM),
        ],
        out_specs=pl.BlockSpec((BLK, H * EHW), lambda i, j: (i, 0)),
        out_shape=jax.ShapeDtypeStruct((N, H * EHW), jnp.float32),
        compiler_params=pltpu.CompilerParams(
            dimension_semantics=("arbitrary", "arbitrary")),
    )(a_mat, b_mat, qext, kext, vext, wpair_l)


def kernel(x, edge_index, edge_attr, aa_embed, w_pos, b_pos, wq, wk, wv, wo,
           w_pair, w1, w2):
    apx = x[:, 0:42:3]
    apy = x[:, 1:42:3]
    apz = x[:, 2:42:3]
    mask = x[:, 45:59]
    aa_col = x[:, 42:43]
    chf = x[:, 44:45].astype(jnp.int32).astype(jnp.float32)
    wx = w_pos[0::3]
    wy = w_pos[1::3]
    wz = w_pos[2::3]
    bpos = b_pos.reshape(1, D)

    h0, a_mat, b_mat = pl.pallas_call(
        _prep_body,
        out_shape=[
            jax.ShapeDtypeStruct((N, D), jnp.float32),
            jax.ShapeDtypeStruct((N, 16), jnp.float32),
            jax.ShapeDtypeStruct((N, 16), jnp.float32),
        ],
    )(apx, apy, apz, mask, aa_col, chf, wx, wy, wz, aa_embed, bpos)

    pre_call = pl.pallas_call(
        _pre_body,
        in_specs=[
            pl.BlockSpec((N, D), lambda: (0, 0)),
            pl.BlockSpec((D, D), lambda: (0, 0)),
            pl.BlockSpec((D, D), lambda: (0, 0)),
            pl.BlockSpec((D, D), lambda: (0, 0)),
            pl.BlockSpec(memory_space=pltpu.SMEM),
        ],
        out_specs=[
            pl.BlockSpec((N, H * EHW), lambda: (0, 0)),
            pl.BlockSpec((N, H * EHW), lambda: (0, 0)),
            pl.BlockSpec((N, H * EHW), lambda: (0, 0)),
        ],
        out_shape=[
            jax.ShapeDtypeStruct((N, H * EHW), jnp.float32),
            jax.ShapeDtypeStruct((N, H * EHW), jnp.float32),
            jax.ShapeDtypeStruct((N, H * EHW), jnp.float32),
        ],
    )

    ff_call = pl.pallas_call(
        _ff_body,
        out_shape=jax.ShapeDtypeStruct((N, D), jnp.float32),
    )

    h = h0
    for l in range(L):
        qext, kext, vext = pre_call(h, wq[l], wk[l], wv[l], w_pair[l])
        acc = _attn_call(a_mat, b_mat, qext, kext, vext, w_pair[l])
        h = ff_call(acc, h, wo[l], w1[l], w2[l])
    return h.reshape(1, N, D)


# speedup vs baseline: 4.3240x; 1.0435x over previous
import math

import jax
import jax.numpy as jnp
from jax.experimental import pallas as pl
from jax.experimental.pallas import tpu as pltpu

N = 2048
D = 128
H = 8
DH = D // H
L = 2
FF = 256
NRBF = 8
NAA = 21
BLK = 256
BLKJ = 1024
IB = N // BLK
JB = N // BLKJ
EHW = 2 * DH

_EXPC = [math.exp(-8.0 * (2.0 * c / (NRBF - 1)) ** 2) for c in range(NRBF)]


def _ln(h):
    mu = jnp.mean(h, axis=-1, keepdims=True)
    hc = h - mu
    var = jnp.mean(hc * hc, axis=-1, keepdims=True)
    return hc / jnp.sqrt(var + 1e-5)


def _prep_body(apx_ref, apy_ref, apz_ref, mask_ref, aa_ref, ch_ref, wx_ref,
               wy_ref, wz_ref, emb_ref, bpos_ref, h0_ref, a_ref, b_ref):
    m = (mask_ref[...] > 0.5).astype(jnp.float32)
    apx = apx_ref[...]
    apy = apy_ref[...]
    apz = apz_ref[...]
    cax = apx[:, 1:2]
    cay = apy[:, 1:2]
    caz = apz[:, 1:2]
    h = jnp.dot((apx - cax) * m, wx_ref[...], preferred_element_type=jnp.float32)
    h += jnp.dot((apy - cay) * m, wy_ref[...], preferred_element_type=jnp.float32)
    h += jnp.dot((apz - caz) * m, wz_ref[...], preferred_element_type=jnp.float32)
    ai = jnp.clip(aa_ref[...].astype(jnp.int32), 0, NAA - 1)
    iota = jax.lax.broadcasted_iota(jnp.int32, (N, NAA), 1)
    onehot = (ai == iota).astype(jnp.float32)
    h += jnp.dot(onehot, emb_ref[...], preferred_element_type=jnp.float32)
    h0_ref[...] = h + bpos_ref[...]
    ch = ch_ref[...]
    n2 = cax * cax + cay * cay + caz * caz
    c2 = ch * ch
    ones = jnp.ones_like(n2)
    zeros = jnp.zeros_like(n2)
    a_ref[...] = jnp.concatenate(
        [-2.0 * cax, -2.0 * cay, -2.0 * caz, n2, ones,
         c2, -2.0 * ch, ones,
         zeros, zeros, zeros, zeros, zeros, zeros, zeros, zeros], axis=1)
    b_ref[...] = jnp.concatenate(
        [cax, cay, caz, ones, n2,
         ones, ch, c2,
         zeros, zeros, zeros, zeros, zeros, zeros, zeros, zeros], axis=1)


def _pre_body(h_ref, wq_ref, wk_ref, wv_ref, wpair_ref,
              qext_ref, kext_ref, vext_ref):
    hn = _ln(h_ref[...])
    q = jnp.dot(hn, wq_ref[...], preferred_element_type=jnp.float32)
    k = jnp.dot(hn, wk_ref[...], preferred_element_type=jnp.float32)
    v = jnp.dot(hn, wv_ref[...], preferred_element_type=jnp.float32)
    i0 = jax.lax.broadcasted_iota(jnp.int32, (D, H), 0) // DH
    i1 = jax.lax.broadcasted_iota(jnp.int32, (D, H), 1)
    sel = (i0 == i1).astype(jnp.float32)
    nq = jnp.sqrt(jnp.dot(q * q, sel, preferred_element_type=jnp.float32))
    nk2 = jnp.dot(k * k, sel, preferred_element_type=jnp.float32)
    kmax = jnp.sqrt(jnp.max(nk2, axis=0, keepdims=True))
    bbs = []
    for h in range(H):
        s = jnp.abs(wpair_ref[NRBF, h])
        for c in range(NRBF):
            s += jnp.abs(wpair_ref[c, h])
        bbs.append(s.reshape(1, 1))
    bb = jnp.concatenate(bbs, axis=1)
    mb = 0.25 * nq * kmax + bb
    j0 = jax.lax.broadcasted_iota(jnp.int32, (D, H * EHW), 0)
    j1 = jax.lax.broadcasted_iota(jnp.int32, (D, H * EHW), 1)
    head = j1 // EHW
    lane = j1 % EHW
    p1 = ((lane < DH) & (j0 == head * DH + lane)).astype(jnp.float32)
    m0 = jax.lax.broadcasted_iota(jnp.int32, (H, H * EHW), 0)
    m1 = jax.lax.broadcasted_iota(jnp.int32, (H, H * EHW), 1)
    pm = ((m1 % EHW == DH) & (m0 == m1 // EHW)).astype(jnp.float32)
    r1 = jax.lax.broadcasted_iota(jnp.int32, (1, H * EHW), 1)
    is_flag = (r1 % EHW == DH)
    negrow = jnp.where(is_flag, -1.0, 0.0)
    qext_ref[...] = (jnp.dot(q, p1 * 0.25, preferred_element_type=jnp.float32)
                     + jnp.dot(mb, pm, preferred_element_type=jnp.float32))
    kext_ref[...] = jnp.dot(k, p1, preferred_element_type=jnp.float32) + negrow
    vext_ref[...] = jnp.dot(v, p1, preferred_element_type=jnp.float32) - negrow


def _attn_body(a_ref, b_ref, qext_ref, kext_ref, vext_ref, wpair_ref, acc_ref):
    jb = pl.program_id(1)

    @pl.when(jb == 0)
    def _reset():
        acc_ref[...] = jnp.zeros((BLK, H * EHW), dtype=jnp.float32)

    a = a_ref[...]
    b = b_ref[...]
    d2 = jax.lax.dot_general(a[:, 0:5], b[:, 0:5], (((1,), (1,)), ((), ())),
                             preferred_element_type=jnp.float32)
    dc2 = jax.lax.dot_general(a[:, 5:8], b[:, 5:8], (((1,), (1,)), ((), ())),
                              preferred_element_type=jnp.float32)
    d2 = jnp.maximum(d2, 0.0) + 1e-8
    d = jnp.sqrt(d2)
    same = jnp.exp(dc2 * -30.0).astype(jnp.bfloat16)
    e0 = jnp.exp(d2 * -8.0).astype(jnp.bfloat16)
    u1 = jnp.exp(d * (32.0 / 7.0))
    u2 = u1 * u1
    u3 = u2 * u1
    u4 = u2 * u2
    u5 = u4 * u1
    u6 = u4 * u2
    u7 = u4 * u3
    ub = [None] + [x.astype(jnp.bfloat16) for x in (u1, u2, u3, u4, u5, u6, u7)]

    qx = qext_ref[...]
    kx = kext_ref[...]
    vt = vext_ref[...]
    for h in range(H):
        sl = slice(h * EHW, (h + 1) * EHW)
        logits = jax.lax.dot_general(qx[:, sl], kx[:, sl],
                                     (((1,), (1,)), ((), ())),
                                     preferred_element_type=jnp.float32)
        s = None
        for c in range(1, NRBF):
            ac = (wpair_ref[c, h] * _EXPC[c]).astype(jnp.bfloat16)
            s = ub[c] * ac if s is None else s + ub[c] * ac
        s += wpair_ref[0, h].astype(jnp.bfloat16)
        bias = s * e0 + same * wpair_ref[NRBF, h].astype(jnp.bfloat16)
        arg = logits + bias.astype(jnp.float32)
        p = jnp.exp(jnp.maximum(arg, -80.0))
        acc_ref[:, sl] += jnp.dot(p, vt[:, sl],
                                  preferred_element_type=jnp.float32)


def _ff_body(acc_ref, h_ref, wo_ref, w1_ref, w2_ref, out_ref):
    acc = acc_ref[...]
    j0 = jax.lax.broadcasted_iota(jnp.int32, (H * EHW, D), 0)
    j1 = jax.lax.broadcasted_iota(jnp.int32, (H * EHW, D), 1)
    head = j1 // DH
    pn = (j0 == head * EHW + (j1 % DH)).astype(jnp.float32)
    pd = (j0 == head * EHW + DH).astype(jnp.float32)
    num = jnp.dot(acc, pn, preferred_element_type=jnp.float32)
    den = jnp.dot(acc, pd, preferred_element_type=jnp.float32)
    o = num / den
    h1 = h_ref[...] + jnp.dot(o, wo_ref[...], preferred_element_type=jnp.float32)
    ffin = jax.nn.relu(jnp.dot(_ln(h1), w1_ref[...], preferred_element_type=jnp.float32))
    out_ref[...] = h1 + jnp.dot(ffin, w2_ref[...], preferred_element_type=jnp.float32)


def _attn_call(a_mat, b_mat, qext, kext, vext, wpair_l):
    return pl.pallas_call(
        _attn_body,
        grid=(IB, JB),
        in_specs=[
            pl.BlockSpec((BLK, 16), lambda i, j: (i, 0)),
            pl.BlockSpec((BLKJ, 16), lambda i, j: (j, 0)),
            pl.BlockSpec((BLK, H * EHW), lambda i, j: (i, 0)),
            pl.BlockSpec((BLKJ, H * EHW), lambda i, j: (j, 0)),
            pl.BlockSpec((BLKJ, H * EHW), lambda i, j: (j, 0)),
            pl.BlockSpec(memory_space=pltpu.SMEM),
        ],
        out_specs=pl.BlockSpec((BLK, H * EHW), lambda i, j: (i, 0)),
        out_shape=jax.ShapeDtypeStruct((N, H * EHW), jnp.float32),
        compiler_params=pltpu.CompilerParams(
            dimension_semantics=("arbitrary", "arbitrary")),
    )(a_mat, b_mat, qext, kext, vext, wpair_l)


def kernel(x, edge_index, edge_attr, aa_embed, w_pos, b_pos, wq, wk, wv, wo,
           w_pair, w1, w2):
    apx = x[:, 0:42:3]
    apy = x[:, 1:42:3]
    apz = x[:, 2:42:3]
    mask = x[:, 45:59]
    aa_col = x[:, 42:43]
    chf = x[:, 44:45].astype(jnp.int32).astype(jnp.float32)
    wx = w_pos[0::3]
    wy = w_pos[1::3]
    wz = w_pos[2::3]
    bpos = b_pos.reshape(1, D)

    h0, a_mat, b_mat = pl.pallas_call(
        _prep_body,
        out_shape=[
            jax.ShapeDtypeStruct((N, D), jnp.float32),
            jax.ShapeDtypeStruct((N, 16), jnp.float32),
            jax.ShapeDtypeStruct((N, 16), jnp.float32),
        ],
    )(apx, apy, apz, mask, aa_col, chf, wx, wy, wz, aa_embed, bpos)

    pre_call = pl.pallas_call(
        _pre_body,
        in_specs=[
            pl.BlockSpec((N, D), lambda: (0, 0)),
            pl.BlockSpec((D, D), lambda: (0, 0)),
            pl.BlockSpec((D, D), lambda: (0, 0)),
            pl.BlockSpec((D, D), lambda: (0, 0)),
            pl.BlockSpec(memory_space=pltpu.SMEM),
        ],
        out_specs=[
            pl.BlockSpec((N, H * EHW), lambda: (0, 0)),
            pl.BlockSpec((N, H * EHW), lambda: (0, 0)),
            pl.BlockSpec((N, H * EHW), lambda: (0, 0)),
        ],
        out_shape=[
            jax.ShapeDtypeStruct((N, H * EHW), jnp.float32),
            jax.ShapeDtypeStruct((N, H * EHW), jnp.float32),
            jax.ShapeDtypeStruct((N, H * EHW), jnp.float32),
        ],
    )

    ff_call = pl.pallas_call(
        _ff_body,
        out_shape=jax.ShapeDtypeStruct((N, D), jnp.float32),
    )

    h = h0
    for l in range(L):
        qext, kext, vext = pre_call(h, wq[l], wk[l], wv[l], w_pair[l])
        acc = _attn_call(a_mat, b_mat, qext, kext, vext, w_pair[l])
        h = ff_call(acc, h, wo[l], w1[l], w2[l])
    return h.reshape(1, N, D)
